# baseline (speedup 1.0000x reference)
import jax
import jax.numpy as jnp
from jax.experimental import pallas as pl
from jax.experimental.pallas import tpu as pltpu

_H = 128
_NA = 3


def _dott(w, x):
    return jax.lax.dot_general(
        w, x, (((1,), (0,)), ((), ())), preferred_element_type=jnp.float32
    )


def _fused_kernel(xs_ref, xa_ref, W1st_ref, W1at_ref, b1_ref, Wgt_ref, bg_ref,
                  W2t_ref, b2_ref,
                  What_ref, bha_ref, Whbt_ref, bhb_ref, q1_ref, q2_ref):
    fc1 = jnp.maximum(
        _dott(W1st_ref[...], xs_ref[...])
        + _dott(W1at_ref[...], xa_ref[...]) + b1_ref[...], 0.0)

    xw = _dott(Wgt_ref[...], fc1)
    m = (xw[:, 0:1] + xw[:, 1:2] + xw[:, 2:3]) * (1.0 / 3.0)
    cols = jax.lax.broadcasted_iota(jnp.int32, (1, xw.shape[1]), 1)
    is_first = pl.program_id(0) == 0
    xw = jnp.where(jnp.logical_and(is_first, cols < _NA), m, xw)

    g = jnp.maximum(xw + bg_ref[...], 0.0) + fc1
    x2 = jnp.maximum(_dott(W2t_ref[...], g) + b2_ref[...], 0.0)

    h = jnp.maximum(_dott(What_ref[...], x2) + bha_ref[...], 0.0)
    q = _dott(Whbt_ref[...], h) + bhb_ref[...]
    qt = jnp.swapaxes(q, 0, 1)
    q1_ref[...] = qt[:, 0:1]
    q2_ref[...] = qt[:, 1:2]


def kernel(s, a, W1, b1, Wg, bg, W2, b2, Wq1a, bq1a, Wq1b, bq1b, Wq2a, bq2a,
           Wq2b, bq2b):
    n = s.shape[1]
    obs = s.shape[2]
    act = a.shape[2]
    in_dim = _NA * (obs + act)

    block = 16384
    grid = (n + block - 1) // block

    xst = s.transpose(0, 2, 1).reshape(_NA * obs, n)
    xat = a.transpose(0, 2, 1).reshape(_NA * act, n)

    W1st = W1[: _NA * obs].T
    W1at = W1[_NA * obs:].T
    Wgt = Wg.T
    W2t = W2.T
    What = jnp.concatenate([Wq1a, Wq2a], axis=1).T
    bha = jnp.concatenate([bq1a, bq2a], axis=0).reshape(2 * _H, 1)
    Whbt = jnp.concatenate(
        [
            jnp.concatenate([Wq1b, jnp.zeros_like(Wq1b)], axis=1),
            jnp.concatenate([jnp.zeros_like(Wq2b), Wq2b], axis=1),
        ],
        axis=0,
    ).T
    bhb = jnp.concatenate([bq1b, bq2b], axis=0).reshape(2, 1)

    b1r = b1.reshape(_H, 1)
    bgr = bg.reshape(_H, 1)
    b2r = b2.reshape(_H, 1)

    q1, q2 = pl.pallas_call(
        _fused_kernel,
        grid=(grid,),
        in_specs=[
            pl.BlockSpec((_NA * obs, block), lambda i: (0, i)),
            pl.BlockSpec((_NA * act, block), lambda i: (0, i)),
            pl.BlockSpec((_H, _NA * obs), lambda i: (0, 0)),
            pl.BlockSpec((_H, _NA * act), lambda i: (0, 0)),
            pl.BlockSpec((_H, 1), lambda i: (0, 0)),
            pl.BlockSpec((_H, _H), lambda i: (0, 0)),
            pl.BlockSpec((_H, 1), lambda i: (0, 0)),
            pl.BlockSpec((_H, _H), lambda i: (0, 0)),
            pl.BlockSpec((_H, 1), lambda i: (0, 0)),
            pl.BlockSpec((2 * _H, _H), lambda i: (0, 0)),
            pl.BlockSpec((2 * _H, 1), lambda i: (0, 0)),
            pl.BlockSpec((2, 2 * _H), lambda i: (0, 0)),
            pl.BlockSpec((2, 1), lambda i: (0, 0)),
        ],
        out_specs=[
            pl.BlockSpec((block, 1), lambda i: (i, 0)),
            pl.BlockSpec((block, 1), lambda i: (i, 0)),
        ],
        out_shape=[
            jax.ShapeDtypeStruct((n, 1), jnp.float32),
            jax.ShapeDtypeStruct((n, 1), jnp.float32),
        ],
        compiler_params=pltpu.CompilerParams(
            dimension_semantics=("parallel",),
            vmem_limit_bytes=100 * 1024 * 1024,
        ),
    )(xst, xat, W1st, W1at, b1r, Wgt, bgr, W2t, b2r, What, bha, Whbt, bhb)

    return (q1, q2)


# speedup vs baseline: 1.0074x; 1.0074x over previous
import jax
import jax.numpy as jnp
from jax.experimental import pallas as pl
from jax.experimental.pallas import tpu as pltpu

_H = 128
_NA = 3


def _dott(w, x):
    return jax.lax.dot_general(
        w, x, (((1,), (0,)), ((), ())), preferred_element_type=jnp.float32
    )


def _fused_kernel(xs_ref, xa_ref, W1st_ref, W1at_ref, b1_ref, Wgt_ref, bg_ref,
                  W2t_ref, b2_ref,
                  What_ref, bha_ref, Whbt_ref, bhb_ref, q1_ref, q2_ref):
    fc1 = jnp.maximum(
        _dott(W1st_ref[...], xs_ref[...])
        + _dott(W1at_ref[...], xa_ref[...]) + b1_ref[...], 0.0)

    xw = _dott(Wgt_ref[...], fc1)
    m = (xw[:, 0:1] + xw[:, 1:2] + xw[:, 2:3]) * (1.0 / 3.0)
    cols = jax.lax.broadcasted_iota(jnp.int32, (1, xw.shape[1]), 1)
    is_first = pl.program_id(0) == 0
    xw = jnp.where(jnp.logical_and(is_first, cols < _NA), m, xw)

    g = jnp.maximum(xw + bg_ref[...], 0.0) + fc1
    x2 = jnp.maximum(_dott(W2t_ref[...], g) + b2_ref[...], 0.0)

    h = jnp.maximum(_dott(What_ref[...], x2) + bha_ref[...], 0.0)
    q = _dott(Whbt_ref[...], h) + bhb_ref[...]
    qt = jnp.swapaxes(q, 0, 1)
    q1_ref[...] = qt[:, 0:1]
    q2_ref[...] = qt[:, 1:2]


def kernel(s, a, W1, b1, Wg, bg, W2, b2, Wq1a, bq1a, Wq1b, bq1b, Wq2a, bq2a,
           Wq2b, bq2b):
    n = s.shape[1]
    obs = s.shape[2]
    act = a.shape[2]
    in_dim = _NA * (obs + act)

    block = 4096
    grid = (n + block - 1) // block

    xst = s.transpose(0, 2, 1).reshape(_NA * obs, n)
    xat = a.transpose(0, 2, 1).reshape(_NA * act, n)

    W1st = W1[: _NA * obs].T
    W1at = W1[_NA * obs:].T
    Wgt = Wg.T
    W2t = W2.T
    What = jnp.concatenate([Wq1a, Wq2a], axis=1).T
    bha = jnp.concatenate([bq1a, bq2a], axis=0).reshape(2 * _H, 1)
    Whbt = jnp.concatenate(
        [
            jnp.concatenate([Wq1b, jnp.zeros_like(Wq1b)], axis=1),
            jnp.concatenate([jnp.zeros_like(Wq2b), Wq2b], axis=1),
        ],
        axis=0,
    ).T
    bhb = jnp.concatenate([bq1b, bq2b], axis=0).reshape(2, 1)

    b1r = b1.reshape(_H, 1)
    bgr = bg.reshape(_H, 1)
    b2r = b2.reshape(_H, 1)

    q1, q2 = pl.pallas_call(
        _fused_kernel,
        grid=(grid,),
        in_specs=[
            pl.BlockSpec((_NA * obs, block), lambda i: (0, i)),
            pl.BlockSpec((_NA * act, block), lambda i: (0, i)),
            pl.BlockSpec((_H, _NA * obs), lambda i: (0, 0)),
            pl.BlockSpec((_H, _NA * act), lambda i: (0, 0)),
            pl.BlockSpec((_H, 1), lambda i: (0, 0)),
            pl.BlockSpec((_H, _H), lambda i: (0, 0)),
            pl.BlockSpec((_H, 1), lambda i: (0, 0)),
            pl.BlockSpec((_H, _H), lambda i: (0, 0)),
            pl.BlockSpec((_H, 1), lambda i: (0, 0)),
            pl.BlockSpec((2 * _H, _H), lambda i: (0, 0)),
            pl.BlockSpec((2 * _H, 1), lambda i: (0, 0)),
            pl.BlockSpec((2, 2 * _H), lambda i: (0, 0)),
            pl.BlockSpec((2, 1), lambda i: (0, 0)),
        ],
        out_specs=[
            pl.BlockSpec((block, 1), lambda i: (i, 0)),
            pl.BlockSpec((block, 1), lambda i: (i, 0)),
        ],
        out_shape=[
            jax.ShapeDtypeStruct((n, 1), jnp.float32),
            jax.ShapeDtypeStruct((n, 1), jnp.float32),
        ],
        compiler_params=pltpu.CompilerParams(
            dimension_semantics=("parallel",),
            vmem_limit_bytes=100 * 1024 * 1024,
        ),
    )(xst, xat, W1st, W1at, b1r, Wgt, bgr, W2t, b2r, What, bha, Whbt, bhb)

    return (q1, q2)


# speedup vs baseline: 1.0358x; 1.0282x over previous
import jax
import jax.numpy as jnp
from jax.experimental import pallas as pl
from jax.experimental.pallas import tpu as pltpu

_H = 128
_NA = 3


def _dott(w, x):
    return jax.lax.dot_general(
        w, x, (((1,), (0,)), ((), ())), preferred_element_type=jnp.float32
    )


def _fused_kernel(xs_ref, xa_ref, W1st_ref, W1at_ref, b1_ref, Wgt_ref, bg_ref,
                  W2t_ref, b2_ref,
                  What_ref, bha_ref, Whbt_ref, bhb_ref, q1_ref, q2_ref):
    fc1 = jnp.maximum(
        _dott(W1st_ref[...], xs_ref[...])
        + _dott(W1at_ref[...], xa_ref[...]) + b1_ref[...], 0.0)

    xw = _dott(Wgt_ref[...], fc1)
    m = (xw[:, 0:1] + xw[:, 1:2] + xw[:, 2:3]) * (1.0 / 3.0)
    cols = jax.lax.broadcasted_iota(jnp.int32, (1, xw.shape[1]), 1)
    is_first = pl.program_id(0) == 0
    xw = jnp.where(jnp.logical_and(is_first, cols < _NA), m, xw)

    g = jnp.maximum(xw + bg_ref[...], 0.0) + fc1
    x2 = jnp.maximum(_dott(W2t_ref[...], g) + b2_ref[...], 0.0)

    h = jnp.maximum(_dott(What_ref[...], x2) + bha_ref[...], 0.0)
    q = _dott(Whbt_ref[...], h) + bhb_ref[...]
    qt = jnp.swapaxes(q, 0, 1)
    q1_ref[...] = qt[:, 0:1]
    q2_ref[...] = qt[:, 1:2]


def kernel(s, a, W1, b1, Wg, bg, W2, b2, Wq1a, bq1a, Wq1b, bq1b, Wq2a, bq2a,
           Wq2b, bq2b):
    n = s.shape[1]
    obs = s.shape[2]
    act = a.shape[2]
    in_dim = _NA * (obs + act)

    block = 8192
    grid = (n + block - 1) // block

    xst = s.transpose(0, 2, 1).reshape(_NA * obs, n)
    xat = a.transpose(0, 2, 1).reshape(_NA * act, n)

    W1st = W1[: _NA * obs].T
    W1at = W1[_NA * obs:].T
    Wgt = Wg.T
    W2t = W2.T
    What = jnp.concatenate([Wq1a, Wq2a], axis=1).T
    bha = jnp.concatenate([bq1a, bq2a], axis=0).reshape(2 * _H, 1)
    Whbt = jnp.concatenate(
        [
            jnp.concatenate([Wq1b, jnp.zeros_like(Wq1b)], axis=1),
            jnp.concatenate([jnp.zeros_like(Wq2b), Wq2b], axis=1),
        ],
        axis=0,
    ).T
    bhb = jnp.concatenate([bq1b, bq2b], axis=0).reshape(2, 1)

    b1r = b1.reshape(_H, 1)
    bgr = bg.reshape(_H, 1)
    b2r = b2.reshape(_H, 1)

    q1, q2 = pl.pallas_call(
        _fused_kernel,
        grid=(grid,),
        in_specs=[
            pl.BlockSpec((_NA * obs, block), lambda i: (0, i)),
            pl.BlockSpec((_NA * act, block), lambda i: (0, i)),
            pl.BlockSpec((_H, _NA * obs), lambda i: (0, 0)),
            pl.BlockSpec((_H, _NA * act), lambda i: (0, 0)),
            pl.BlockSpec((_H, 1), lambda i: (0, 0)),
            pl.BlockSpec((_H, _H), lambda i: (0, 0)),
            pl.BlockSpec((_H, 1), lambda i: (0, 0)),
            pl.BlockSpec((_H, _H), lambda i: (0, 0)),
            pl.BlockSpec((_H, 1), lambda i: (0, 0)),
            pl.BlockSpec((2 * _H, _H), lambda i: (0, 0)),
            pl.BlockSpec((2 * _H, 1), lambda i: (0, 0)),
            pl.BlockSpec((2, 2 * _H), lambda i: (0, 0)),
            pl.BlockSpec((2, 1), lambda i: (0, 0)),
        ],
        out_specs=[
            pl.BlockSpec((block, 1), lambda i: (i, 0)),
            pl.BlockSpec((block, 1), lambda i: (i, 0)),
        ],
        out_shape=[
            jax.ShapeDtypeStruct((n, 1), jnp.float32),
            jax.ShapeDtypeStruct((n, 1), jnp.float32),
        ],
        compiler_params=pltpu.CompilerParams(
            dimension_semantics=("parallel",),
            vmem_limit_bytes=100 * 1024 * 1024,
        ),
    )(xst, xat, W1st, W1at, b1r, Wgt, bgr, W2t, b2r, What, bha, Whbt, bhb)

    return (q1, q2)
